# speedup vs baseline: 1.9036x; 1.9036x over previous
import functools

import jax
import jax.numpy as jnp
from jax import lax
from jax.experimental import pallas as pl
from jax.experimental.pallas import tpu as pltpu
from jax.experimental.pallas import tpu_sc as plsc

EMBED_DIM = 128
TABLE_ROWS = 1000
BATCH = 16384

NUM_CORES = 2
NUM_SUBCORES = 16
LANES = 16
NUM_WORKERS = NUM_CORES * NUM_SUBCORES
B_PER_W = BATCH // NUM_WORKERS


def _gather_body(t_hbm, embed_hbm, out_hbm, idx_v, rows_v, sem):
    wid = lax.axis_index("s") * NUM_CORES + lax.axis_index("c")
    base = wid * B_PER_W

    pltpu.sync_copy(t_hbm.at[pl.ds(base, B_PER_W)], idx_v)

    for i in range(B_PER_W // LANES):
        sl = pl.ds(i * LANES, LANES)
        v = idx_v[sl] - 1
        idx_v[sl] = jnp.where(v < 0, v + TABLE_ROWS, v)

    pltpu.async_copy(embed_hbm.at[idx_v], rows_v, sem).wait()

    pltpu.sync_copy(rows_v, out_hbm.at[pl.ds(base, B_PER_W)])


@jax.jit
def kernel(t, embed):
    run = pl.kernel(
        _gather_body,
        mesh=plsc.VectorSubcoreMesh(core_axis_name="c", subcore_axis_name="s"),
        out_type=jax.ShapeDtypeStruct((BATCH, EMBED_DIM), jnp.float32),
        scratch_types=[
            pltpu.VMEM((B_PER_W,), jnp.int32),
            pltpu.VMEM((B_PER_W, EMBED_DIM), jnp.float32),
            pltpu.SemaphoreType.DMA,
        ],
    )
    return run(t.astype(jnp.int32), embed.astype(jnp.float32))
